# baseline (speedup 1.0000x reference)
import jax
import jax.numpy as jnp
from jax.experimental import pallas as pl
from jax.experimental.pallas import tpu as pltpu

N = 5000
S = 16
H = 64
GD = 4 * H
B = 1000
GRID = N // B


def _lstm_block_kernel(x_ref, wih0_ref, whh0_ref, b0_ref,
                       wih1_ref, whh1_ref, b1_ref,
                       hw_ref, hb_ref, out_ref, hs_ref):
    x = x_ref[...].reshape(B * S, H)
    g0 = (jnp.dot(x, wih0_ref[...], preferred_element_type=jnp.float32)
          + b0_ref[...]).reshape(B, S, GD)

    whh0 = whh0_ref[...]
    h = jnp.zeros((B, H), jnp.float32)
    c = jnp.zeros((B, H), jnp.float32)
    for t in range(S):
        gates = g0[:, t, :] + jnp.dot(h, whh0,
                                      preferred_element_type=jnp.float32)
        i = jax.nn.sigmoid(gates[:, 0:H])
        f = jax.nn.sigmoid(gates[:, H:2 * H])
        g = jnp.tanh(gates[:, 2 * H:3 * H])
        o = jax.nn.sigmoid(gates[:, 3 * H:4 * H])
        c = f * c + i * g
        h = o * jnp.tanh(c)
        hs_ref[:, t, :] = h

    x1 = hs_ref[...].reshape(B * S, H)
    g1 = (jnp.dot(x1, wih1_ref[...], preferred_element_type=jnp.float32)
          + b1_ref[...]).reshape(B, S, GD)

    whh1 = whh1_ref[...]
    h = jnp.zeros((B, H), jnp.float32)
    c = jnp.zeros((B, H), jnp.float32)
    for t in range(S):
        gates = g1[:, t, :] + jnp.dot(h, whh1,
                                      preferred_element_type=jnp.float32)
        i = jax.nn.sigmoid(gates[:, 0:H])
        f = jax.nn.sigmoid(gates[:, H:2 * H])
        g = jnp.tanh(gates[:, 2 * H:3 * H])
        o = jax.nn.sigmoid(gates[:, 3 * H:4 * H])
        c = f * c + i * g
        h = o * jnp.tanh(c)

    out_ref[...] = (jnp.dot(h, hw_ref[...],
                            preferred_element_type=jnp.float32)
                    + hb_ref[...])


def kernel(data, label, proposals, classes,
           w_ih_0, w_hh_0, b_ih_0, b_hh_0,
           w_ih_1, w_hh_1, b_ih_1, b_hh_1,
           cls_w, cls_b, bbox_w, bbox_b):
    f32 = jnp.float32
    wih0T = w_ih_0.T
    whh0T = w_hh_0.T
    b0 = (b_ih_0 + b_hh_0).reshape(1, GD)
    wih1T = w_ih_1.T
    whh1T = w_hh_1.T
    b1 = (b_ih_1 + b_hh_1).reshape(1, GD)
    hw = jnp.concatenate([cls_w, bbox_w, jnp.zeros((1, H), f32)], axis=0).T
    hb = jnp.concatenate([cls_b, bbox_b, jnp.zeros((1,), f32)]).reshape(1, 8)

    out = pl.pallas_call(
        _lstm_block_kernel,
        grid=(GRID,),
        in_specs=[
            pl.BlockSpec((B, S, H), lambda i: (i, 0, 0)),
            pl.BlockSpec((H, GD), lambda i: (0, 0)),
            pl.BlockSpec((H, GD), lambda i: (0, 0)),
            pl.BlockSpec((1, GD), lambda i: (0, 0)),
            pl.BlockSpec((H, GD), lambda i: (0, 0)),
            pl.BlockSpec((H, GD), lambda i: (0, 0)),
            pl.BlockSpec((1, GD), lambda i: (0, 0)),
            pl.BlockSpec((H, 8), lambda i: (0, 0)),
            pl.BlockSpec((1, 8), lambda i: (0, 0)),
        ],
        out_specs=pl.BlockSpec((B, 8), lambda i: (i, 0)),
        out_shape=jax.ShapeDtypeStruct((N, 8), f32),
        scratch_shapes=[pltpu.VMEM((B, S, H), f32)],
        compiler_params=pltpu.CompilerParams(
            dimension_semantics=("parallel",)),
    )(proposals, wih0T, whh0T, b0, wih1T, whh1T, b1, hw, hb)

    cls_feat = out[:, :5]
    bbox_feat = out[:, 5:7]
    return (cls_feat, bbox_feat, jnp.float32(0.0), jnp.float32(0.0))


# speedup vs baseline: 2.8166x; 2.8166x over previous
import jax
import jax.numpy as jnp
from jax.experimental import pallas as pl
from jax.experimental.pallas import tpu as pltpu

N = 5000
NP = 5120
S = 16
H = 64
GD = 4 * H
B = 512
GRID = NP // B


def _lstm_block_kernel(x_ref, wih0_ref, whh0_ref, b0_ref,
                       wih1_ref, whh1_ref, b1_ref,
                       hw_ref, out_ref, hs_ref):
    x = x_ref[0]
    g0 = jnp.dot(wih0_ref[...], x, preferred_element_type=jnp.float32)

    whh0 = whh0_ref[...]
    b0 = b0_ref[...]
    h = jnp.zeros((H, B), jnp.float32)
    c = jnp.zeros((H, B), jnp.float32)
    for t in range(S):
        gates = (g0[:, t * B:(t + 1) * B] + b0
                 + jnp.dot(whh0, h, preferred_element_type=jnp.float32))
        i = jax.nn.sigmoid(gates[0:H])
        f = jax.nn.sigmoid(gates[H:2 * H])
        g = jnp.tanh(gates[2 * H:3 * H])
        o = jax.nn.sigmoid(gates[3 * H:4 * H])
        c = f * c + i * g
        h = o * jnp.tanh(c)
        hs_ref[:, t * B:(t + 1) * B] = h

    g1 = jnp.dot(wih1_ref[...], hs_ref[...],
                 preferred_element_type=jnp.float32)

    whh1 = whh1_ref[...]
    b1 = b1_ref[...]
    h = jnp.zeros((H, B), jnp.float32)
    c = jnp.zeros((H, B), jnp.float32)
    for t in range(S):
        gates = (g1[:, t * B:(t + 1) * B] + b1
                 + jnp.dot(whh1, h, preferred_element_type=jnp.float32))
        i = jax.nn.sigmoid(gates[0:H])
        f = jax.nn.sigmoid(gates[H:2 * H])
        g = jnp.tanh(gates[2 * H:3 * H])
        o = jax.nn.sigmoid(gates[3 * H:4 * H])
        c = f * c + i * g
        h = o * jnp.tanh(c)

    out_ref[...] = jnp.dot(hw_ref[...], h,
                           preferred_element_type=jnp.float32)


def kernel(data, label, proposals, classes,
           w_ih_0, w_hh_0, b_ih_0, b_hh_0,
           w_ih_1, w_hh_1, b_ih_1, b_hh_1,
           cls_w, cls_b, bbox_w, bbox_b):
    f32 = jnp.float32
    xp = jnp.pad(proposals, ((0, NP - N), (0, 0), (0, 0)))
    xp = xp.reshape(GRID, B, S, H).transpose(0, 3, 2, 1).reshape(GRID, H, S * B)

    b0 = jnp.tile((b_ih_0 + b_hh_0).reshape(GD, 1), (1, B))
    b1 = jnp.tile((b_ih_1 + b_hh_1).reshape(GD, 1), (1, B))
    hw = jnp.concatenate([cls_w, bbox_w, jnp.zeros((1, H), f32)], axis=0)

    out = pl.pallas_call(
        _lstm_block_kernel,
        grid=(GRID,),
        in_specs=[
            pl.BlockSpec((1, H, S * B), lambda i: (i, 0, 0)),
            pl.BlockSpec((GD, H), lambda i: (0, 0)),
            pl.BlockSpec((GD, H), lambda i: (0, 0)),
            pl.BlockSpec((GD, B), lambda i: (0, 0)),
            pl.BlockSpec((GD, H), lambda i: (0, 0)),
            pl.BlockSpec((GD, H), lambda i: (0, 0)),
            pl.BlockSpec((GD, B), lambda i: (0, 0)),
            pl.BlockSpec((8, H), lambda i: (0, 0)),
        ],
        out_specs=pl.BlockSpec((8, B), lambda i: (0, i)),
        out_shape=jax.ShapeDtypeStruct((8, NP), f32),
        scratch_shapes=[pltpu.VMEM((H, S * B), f32)],
        compiler_params=pltpu.CompilerParams(
            dimension_semantics=("parallel",)),
    )(xp, w_ih_0, w_hh_0, b0, w_ih_1, w_hh_1, b1, hw)

    outT = out.T[:N]
    cls_feat = outT[:, :5] + cls_b
    bbox_feat = outT[:, 5:7] + bbox_b
    return (cls_feat, bbox_feat, jnp.float32(0.0), jnp.float32(0.0))


# speedup vs baseline: 3.1496x; 1.1182x over previous
import jax
import jax.numpy as jnp
from jax.experimental import pallas as pl
from jax.experimental.pallas import tpu as pltpu

N = 5000
NP = 5120
S = 16
H = 64
GD = 4 * H
B = 2560
GRID = NP // B


def _lstm_block_kernel(x_ref, w0_ref, b0_ref, w1_ref, b1_ref,
                       hw_ref, out_ref):
    x = x_ref[0]
    w0 = w0_ref[...]
    b0 = b0_ref[...]
    w1 = w1_ref[...]
    b1 = b1_ref[...]

    z = jnp.zeros((H, B), jnp.float32)
    h0, c0, h1, c1 = z, z, z, z

    def cell(w, b, xt, h, c):
        gates = b + jnp.dot(w, jnp.concatenate([xt, h], axis=0),
                            preferred_element_type=jnp.float32)
        i = jax.nn.sigmoid(gates[0:H])
        f = jax.nn.sigmoid(gates[H:2 * H])
        g = jnp.tanh(gates[2 * H:3 * H])
        o = jax.nn.sigmoid(gates[3 * H:4 * H])
        c = f * c + i * g
        h = o * jnp.tanh(c)
        return h, c

    for t in range(S):
        h0, c0 = cell(w0, b0, x[:, t * B:(t + 1) * B], h0, c0)
        h1, c1 = cell(w1, b1, h0, h1, c1)

    out_ref[...] = jnp.dot(hw_ref[...], h1,
                           preferred_element_type=jnp.float32)


def kernel(data, label, proposals, classes,
           w_ih_0, w_hh_0, b_ih_0, b_hh_0,
           w_ih_1, w_hh_1, b_ih_1, b_hh_1,
           cls_w, cls_b, bbox_w, bbox_b):
    f32 = jnp.float32
    xp = jnp.pad(proposals, ((0, NP - N), (0, 0), (0, 0)))
    xp = xp.reshape(GRID, B, S, H).transpose(0, 3, 2, 1).reshape(GRID, H, S * B)

    w0 = jnp.concatenate([w_ih_0, w_hh_0], axis=1)
    w1 = jnp.concatenate([w_ih_1, w_hh_1], axis=1)
    b0 = jnp.tile((b_ih_0 + b_hh_0).reshape(GD, 1), (1, B))
    b1 = jnp.tile((b_ih_1 + b_hh_1).reshape(GD, 1), (1, B))
    hw = jnp.concatenate([cls_w, bbox_w, jnp.zeros((1, H), f32)], axis=0)

    out = pl.pallas_call(
        _lstm_block_kernel,
        grid=(GRID,),
        in_specs=[
            pl.BlockSpec((1, H, S * B), lambda i: (i, 0, 0)),
            pl.BlockSpec((GD, 2 * H), lambda i: (0, 0)),
            pl.BlockSpec((GD, B), lambda i: (0, 0)),
            pl.BlockSpec((GD, 2 * H), lambda i: (0, 0)),
            pl.BlockSpec((GD, B), lambda i: (0, 0)),
            pl.BlockSpec((8, H), lambda i: (0, 0)),
        ],
        out_specs=pl.BlockSpec((8, B), lambda i: (0, i)),
        out_shape=jax.ShapeDtypeStruct((8, NP), f32),
        compiler_params=pltpu.CompilerParams(
            dimension_semantics=("parallel",)),
    )(xp, w0, b0, w1, b1, hw)

    outT = out.T[:N]
    cls_feat = outT[:, :5] + cls_b
    bbox_feat = outT[:, 5:7] + bbox_b
    return (cls_feat, bbox_feat, jnp.float32(0.0), jnp.float32(0.0))
